# baseline (speedup 1.0000x reference)
import jax
import jax.numpy as jnp
from jax import lax
from jax.experimental import pallas as pl
from jax.experimental.pallas import tpu as pltpu

_B = 16
_TOPK = 60
_THRESHOLD = 0.15
_NMS_THRESHOLD = 0.05
_NMS_TOPK = 20
_N = 24 * 24 * 24
_ROWS = 108
_LANES = 128
_NEG = -1e30
_HI = jax.lax.Precision.HIGHEST


def _body(cls_ref, shp_ref, off_ref, out_ref):
    riota = lax.broadcasted_iota(jnp.int32, (_ROWS, _LANES), 0)
    liota = lax.broadcasted_iota(jnp.int32, (_ROWS, _LANES), 1)
    fiota = riota * _LANES + liota
    k60c = lax.broadcasted_iota(jnp.int32, (_TOPK, 1), 0)
    r60 = lax.broadcasted_iota(jnp.int32, (_TOPK, _TOPK), 0)
    l60 = lax.broadcasted_iota(jnp.int32, (_TOPK, _TOPK), 1)
    eye = (r60 == l60).astype(jnp.float32)
    i60r = lax.broadcasted_iota(jnp.int32, (1, _TOPK), 1)
    r60_8 = lax.broadcasted_iota(jnp.int32, (_TOPK, 8), 0)
    r20 = lax.broadcasted_iota(jnp.int32, (_NMS_TOPK, 8), 0)
    l8 = lax.broadcasted_iota(jnp.int32, (1, 8), 1)
    zc = jnp.zeros((_TOPK, 1), jnp.float32)
    zi = jnp.zeros((_TOPK, 1), jnp.int32)
    neg1 = jnp.float32(-1.0)

    def tr(v):
        return lax.dot_general(v, eye, (((0,), (0,)), ((), ())),
                               precision=_HI)

    padV = jnp.full((20, _LANES), -1.0, jnp.float32)
    padI = jnp.full((20, _LANES), _N, jnp.int32)
    li128 = lax.broadcasted_iota(jnp.int32, (128, _LANES), 1)
    rowasc = lax.broadcasted_iota(jnp.int32, (128, 1), 0) >= 64

    def xshuf(x, j):
        li = lax.broadcasted_iota(jnp.int32, x.shape, 1)
        lo = pltpu.roll(x, _LANES - j, 1)
        hi = pltpu.roll(x, j, 1)
        return jnp.where((li & j) == 0, lo, hi)

    def stage(V, I, j, flip):
        pV = xshuf(V, j)
        pI = xshuf(I, j)
        R = (V > pV) | ((V == pV) & (I < pI))
        keep = R ^ flip
        return jnp.where(keep, V, pV), jnp.where(keep, I, pI)

    vals_t, idxs_t = [], []
    for s in range(_B):
        V = jnp.concatenate([1.0 / (1.0 + jnp.exp(-cls_ref[s])), padV],
                            axis=0)
        I = jnp.concatenate([fiota, padI], axis=0)
        for k in (2, 4, 8, 16, 32, 64, 128):
            asc = ((li128 & k) != 0) ^ rowasc
            jj = k // 2
            while jj >= 1:
                V, I = stage(V, I, jj, asc ^ ((li128 & jj) != 0))
                jj //= 2
        for h in (64, 32, 16, 8, 4, 2, 1):
            Av, Bv, Ai, Bi = V[:h], V[h:2 * h], I[:h], I[h:2 * h]
            R = (Av > Bv) | ((Av == Bv) & (Ai < Bi))
            V = jnp.where(R, Av, Bv)
            I = jnp.where(R, Ai, Bi)
            ra = lax.broadcasted_iota(jnp.int32, (h, 1), 0) >= ((h + 1) // 2)
            lih = lax.broadcasted_iota(jnp.int32, (h, _LANES), 1)
            for j2 in (64, 32, 16, 8, 4, 2, 1):
                V, I = stage(V, I, j2, ra ^ ((lih & j2) != 0))
        sv = V[:, :_TOPK]
        si = I[:, :_TOPK].astype(jnp.float32)
        vals_t.append(lax.dot_general(eye, sv, (((1,), (1,)), ((), ())),
                                      precision=_HI))
        idxs_t.append(lax.dot_general(eye, si, (((1,), (1,)), ((), ())),
                                      precision=_HI).astype(jnp.int32))

    dets, ious, ts_rs, ts_cs, bts, a_r0, a_c0 = [], [], [], [], [], [], []
    for s in range(_B):
        vals_c, idxs_c = vals_t[s], idxs_t[s]
        hi_c = idxs_c // _LANES
        lo_c = idxs_c % _LANES
        oh_hi = (hi_c == lax.broadcasted_iota(jnp.int32, (_TOPK, _ROWS), 1)
                 ).astype(jnp.float32)
        oh_lo = (lo_c == lax.broadcasted_iota(jnp.int32, (_TOPK, _LANES), 1)
                 ).astype(jnp.float32)

        def gather(plane, oh_hi=oh_hi, oh_lo=oh_lo):
            rows = lax.dot_general(oh_hi, plane, (((1,), (0,)), ((), ())),
                                   precision=_HI)
            return jnp.sum(rows * oh_lo, axis=1, keepdims=True)

        goz = gather(off_ref[s, 0])
        goy = gather(off_ref[s, 1])
        gox = gather(off_ref[s, 2])
        gsz = gather(shp_ref[s, 0])
        gsy = gather(shp_ref[s, 1])
        gsx = gather(shp_ref[s, 2])

        az = (idxs_c // 576).astype(jnp.float32)
        ay = ((idxs_c // 24) % 24).astype(jnp.float32)
        ax = (idxs_c % 24).astype(jnp.float32)

        cz = (az + goz) * 4.0
        cy = (ay + goy) * 4.0
        cx = (ax + gox) * 4.0
        sz = (2.0 * gsz) * 4.0
        sy = (2.0 * gsy) * 4.0
        sx = (2.0 * gsx) * 4.0

        valid_c = vals_c > _THRESHOLD
        det = jnp.concatenate(
            [jnp.where(valid_c, 1.0, neg1),
             jnp.where(valid_c, vals_c, neg1),
             jnp.where(valid_c, cz, neg1),
             jnp.where(valid_c, cy, neg1),
             jnp.where(valid_c, cx, neg1),
             jnp.where(valid_c, sz, neg1),
             jnp.where(valid_c, sy, neg1),
             jnp.where(valid_c, sx, neg1)], axis=1)

        def pair(c_c, s_c):
            c_r, s_r = tr(c_c), tr(s_c)
            lo_cc, hi_cc = c_c - s_c * 0.5, c_c + s_c * 0.5
            lo_rr, hi_rr = c_r - s_r * 0.5, c_r + s_r * 0.5
            return jnp.maximum(
                jnp.minimum(hi_cc, hi_rr) - jnp.maximum(lo_cc, lo_rr), 0.0)

        inter = (pair(cz, sz) * pair(cy, sy)) * pair(cx, sx)
        vol_c = (jnp.maximum(sz, 0.0) * jnp.maximum(sy, 0.0)
                 ) * jnp.maximum(sx, 0.0)
        vol_r = tr(vol_c)
        union = (vol_c + vol_r) - inter
        iou = inter / jnp.maximum(union, 1e-6)
        mth = (iou >= _NMS_THRESHOLD).astype(jnp.float32)

        ts_c = vals_c
        ts_r = tr(ts_c)
        bt = ((ts_c > ts_r) | ((ts_c == ts_r) & (r60 < l60))
              ).astype(jnp.float32)

        dets.append(det)
        ious.append(mth)
        ts_rs.append(ts_r)
        ts_cs.append(ts_c)
        bts.append(bt)
        a_r0.append(tr(valid_c.astype(jnp.float32)))
        a_c0.append(valid_c.astype(jnp.float32))

    def nms(t, carry):
        ar, ac, outs = carry
        nar, nac, nout = [], [], []
        for s in range(_B):
            alive_r = ar[s] > 0.5
            alive_c = ac[s] > 0.5
            ms = jnp.where(alive_r, ts_rs[s], _NEG)
            m = jnp.max(ms)
            i = jnp.min(jnp.where(ms == m, i60r, _TOPK))
            any_alive = jnp.any(alive_r)
            oh = (i60r == i).astype(jnp.float32)
            mrow = lax.dot_general(oh, ious[s], (((1,), (0,)), ((), ())))
            mcol = lax.dot_general(ious[s], oh, (((1,), (1,)), ((), ())))
            matched_r = alive_r & (mrow > 0.5)
            matched_c = alive_c & (mcol > 0.5)
            m_count = jnp.sum(matched_r.astype(jnp.int32))
            rank_c = lax.dot_general(bts[s], matched_c.astype(jnp.float32),
                                     (((1,), (0,)), ((), ())))
            sel_c = matched_c & (rank_c < 6.5)
            cnt = jnp.maximum(jnp.minimum(m_count, 7).astype(jnp.float32),
                              1.0)
            sum_det = lax.dot_general(sel_c.astype(jnp.float32), dets[s],
                                      (((0,), (0,)), ((), ())),
                                      precision=_HI)
            det_i = lax.dot_general(oh, dets[s], (((1,), (0,)), ((), ())),
                                    precision=_HI)
            avg = sum_det / cnt
            avg = jnp.where(l8 == 0, 1.0, jnp.where(l8 == 1, m, avg))
            row = jnp.where(jnp.minimum(m_count, 7) > 1, avg, det_i)
            row = jnp.where(any_alive, row, neg1)
            nout.append(jnp.where(r20 == t, row, outs[s]))
            nar.append((alive_r & (~matched_r)).astype(jnp.float32))
            nac.append((alive_c & (~matched_c)).astype(jnp.float32))
        return tuple(nar), tuple(nac), tuple(nout)

    z20 = jnp.zeros((_NMS_TOPK, 8), jnp.float32)
    _, _, outs = lax.fori_loop(
        0, _NMS_TOPK, nms, (tuple(a_r0), tuple(a_c0), (z20,) * _B))

    pad = jnp.full((_TOPK - _NMS_TOPK, 8), -1.0, jnp.float32)
    for s in range(_B):
        out_ref[s] = jnp.concatenate([outs[s], pad], axis=0)


@jax.jit
def kernel(Cls, Shape, Offset):
    B = Cls.shape[0]
    cls3 = Cls.reshape(B, _ROWS, _LANES)
    shp4 = Shape.reshape(B, 3, _ROWS, _LANES)
    off4 = Offset.reshape(B, 3, _ROWS, _LANES)
    return pl.pallas_call(
        _body,
        out_shape=jax.ShapeDtypeStruct((B, _TOPK, 8), jnp.float32),
    )(cls3, shp4, off4)


# speedup vs baseline: 1.1009x; 1.1009x over previous
import jax
import jax.numpy as jnp
from jax import lax
from jax.experimental import pallas as pl
from jax.experimental.pallas import tpu as pltpu

_B = 16
_TOPK = 60
_THRESHOLD = 0.15
_NMS_THRESHOLD = 0.05
_NMS_TOPK = 20
_N = 24 * 24 * 24
_ROWS = 108
_LANES = 128
_NEG = -1e30
_HI = jax.lax.Precision.HIGHEST


def _body(cls_ref, shp_ref, off_ref, out_ref):
    riota = lax.broadcasted_iota(jnp.int32, (_ROWS, _LANES), 0)
    liota = lax.broadcasted_iota(jnp.int32, (_ROWS, _LANES), 1)
    fiota = riota * _LANES + liota
    k60c = lax.broadcasted_iota(jnp.int32, (_TOPK, 1), 0)
    r60 = lax.broadcasted_iota(jnp.int32, (_TOPK, _TOPK), 0)
    l60 = lax.broadcasted_iota(jnp.int32, (_TOPK, _TOPK), 1)
    eye = (r60 == l60).astype(jnp.float32)
    i60r = lax.broadcasted_iota(jnp.int32, (1, _TOPK), 1)
    r60_8 = lax.broadcasted_iota(jnp.int32, (_TOPK, 8), 0)
    r20 = lax.broadcasted_iota(jnp.int32, (_NMS_TOPK, 8), 0)
    l8 = lax.broadcasted_iota(jnp.int32, (1, 8), 1)
    zc = jnp.zeros((_TOPK, 1), jnp.float32)
    zi = jnp.zeros((_TOPK, 1), jnp.int32)
    neg1 = jnp.float32(-1.0)

    def tr(v):
        return lax.dot_general(v, eye, (((0,), (0,)), ((), ())),
                               precision=_HI)

    padV = jnp.full((20, _LANES), -1.0, jnp.float32)
    padI = jnp.full((20, _LANES), _N, jnp.int32)
    li128 = lax.broadcasted_iota(jnp.int32, (128, _LANES), 1)
    rowasc = lax.broadcasted_iota(jnp.int32, (128, 1), 0) >= 64

    def xshuf(x, j):
        li = lax.broadcasted_iota(jnp.int32, x.shape, 1)
        lo = pltpu.roll(x, _LANES - j, 1)
        hi = pltpu.roll(x, j, 1)
        return jnp.where((li & j) == 0, lo, hi)

    def stage(V, I, j, flip):
        pV = xshuf(V, j)
        pI = xshuf(I, j)
        R = (V > pV) | ((V == pV) & (I < pI))
        keep = R ^ flip
        return jnp.where(keep, V, pV), jnp.where(keep, I, pI)

    vals_t, idxs_t = [], []
    for s in range(_B):
        V = jnp.concatenate([1.0 / (1.0 + jnp.exp(-cls_ref[s])), padV],
                            axis=0)
        I = jnp.concatenate([fiota, padI], axis=0)
        for k in (2, 4, 8, 16, 32, 64, 128):
            asc = ((li128 & k) != 0) ^ rowasc
            jj = k // 2
            while jj >= 1:
                V, I = stage(V, I, jj, asc ^ ((li128 & jj) != 0))
                jj //= 2
        for h in (64, 32, 16, 8, 4, 2, 1):
            Av, Bv, Ai, Bi = V[:h], V[h:2 * h], I[:h], I[h:2 * h]
            R = (Av > Bv) | ((Av == Bv) & (Ai < Bi))
            V = jnp.where(R, Av, Bv)
            I = jnp.where(R, Ai, Bi)
            ra = lax.broadcasted_iota(jnp.int32, (h, 1), 0) >= ((h + 1) // 2)
            lih = lax.broadcasted_iota(jnp.int32, (h, _LANES), 1)
            for j2 in (64, 32, 16, 8, 4, 2, 1):
                V, I = stage(V, I, j2, ra ^ ((lih & j2) != 0))
        sv = V[:, :_TOPK]
        si = I[:, :_TOPK].astype(jnp.float32)
        vals_t.append(lax.dot_general(eye, sv, (((1,), (1,)), ((), ())),
                                      precision=_HI))
        idxs_t.append(lax.dot_general(eye, si, (((1,), (1,)), ((), ())),
                                      precision=_HI).astype(jnp.int32))

    dets, ious, ts_rs, ts_cs, bts, a_r0, a_c0 = [], [], [], [], [], [], []
    for s in range(_B):
        vals_c, idxs_c = vals_t[s], idxs_t[s]
        hi_c = idxs_c // _LANES
        lo_c = idxs_c % _LANES
        oh_hi = (hi_c == lax.broadcasted_iota(jnp.int32, (_TOPK, _ROWS), 1)
                 ).astype(jnp.float32)
        oh_lo = (lo_c == lax.broadcasted_iota(jnp.int32, (_TOPK, _LANES), 1)
                 ).astype(jnp.float32)

        def gather(plane, oh_hi=oh_hi, oh_lo=oh_lo):
            rows = lax.dot_general(oh_hi, plane, (((1,), (0,)), ((), ())),
                                   precision=_HI)
            return jnp.sum(rows * oh_lo, axis=1, keepdims=True)

        goz = gather(off_ref[s, 0])
        goy = gather(off_ref[s, 1])
        gox = gather(off_ref[s, 2])
        gsz = gather(shp_ref[s, 0])
        gsy = gather(shp_ref[s, 1])
        gsx = gather(shp_ref[s, 2])

        az = (idxs_c // 576).astype(jnp.float32)
        ay = ((idxs_c // 24) % 24).astype(jnp.float32)
        ax = (idxs_c % 24).astype(jnp.float32)

        cz = (az + goz) * 4.0
        cy = (ay + goy) * 4.0
        cx = (ax + gox) * 4.0
        sz = (2.0 * gsz) * 4.0
        sy = (2.0 * gsy) * 4.0
        sx = (2.0 * gsx) * 4.0

        valid_c = vals_c > _THRESHOLD
        det = jnp.concatenate(
            [jnp.where(valid_c, 1.0, neg1),
             jnp.where(valid_c, vals_c, neg1),
             jnp.where(valid_c, cz, neg1),
             jnp.where(valid_c, cy, neg1),
             jnp.where(valid_c, cx, neg1),
             jnp.where(valid_c, sz, neg1),
             jnp.where(valid_c, sy, neg1),
             jnp.where(valid_c, sx, neg1)], axis=1)

        def pair(c_c, s_c):
            c_r, s_r = tr(c_c), tr(s_c)
            lo_cc, hi_cc = c_c - s_c * 0.5, c_c + s_c * 0.5
            lo_rr, hi_rr = c_r - s_r * 0.5, c_r + s_r * 0.5
            return jnp.maximum(
                jnp.minimum(hi_cc, hi_rr) - jnp.maximum(lo_cc, lo_rr), 0.0)

        inter = (pair(cz, sz) * pair(cy, sy)) * pair(cx, sx)
        vol_c = (jnp.maximum(sz, 0.0) * jnp.maximum(sy, 0.0)
                 ) * jnp.maximum(sx, 0.0)
        vol_r = tr(vol_c)
        union = (vol_c + vol_r) - inter
        iou = inter / jnp.maximum(union, 1e-6)
        mth = (iou >= _NMS_THRESHOLD).astype(jnp.float32)

        ts_c = vals_c
        ts_r = tr(ts_c)
        bt = ((ts_c > ts_r) | ((ts_c == ts_r) & (r60 < l60))
              ).astype(jnp.float32)

        dets.append(det)
        ious.append(mth)
        ts_rs.append(ts_r)
        ts_cs.append(ts_c)
        bts.append(bt)
        a_r0.append(tr(valid_c.astype(jnp.float32)))
        a_c0.append(valid_c.astype(jnp.float32))

    def nms(t, carry):
        ar, ac, outs = carry
        nar, nac, nout = [], [], []
        for s in range(_B):
            alive_r = ar[s] > 0.5
            alive_c = ac[s] > 0.5
            ms = jnp.where(alive_r, ts_rs[s], _NEG)
            m = jnp.max(ms)
            i = jnp.min(jnp.where(ms == m, i60r, _TOPK))
            any_alive = jnp.any(alive_r)
            mrow = jnp.sum(jnp.where(r60 == i, ious[s], 0.0), axis=0,
                           keepdims=True)
            mcol = jnp.sum(jnp.where(l60 == i, ious[s], 0.0), axis=1,
                           keepdims=True)
            matched_r = alive_r & (mrow > 0.5)
            matched_c = alive_c & (mcol > 0.5)
            m_count = jnp.sum(matched_r.astype(jnp.int32))
            rank_c = lax.dot_general(bts[s], matched_c.astype(jnp.float32),
                                     (((1,), (0,)), ((), ())), precision=_HI)
            sel_c = matched_c & (rank_c < 6.5)
            cnt = jnp.maximum(jnp.minimum(m_count, 7).astype(jnp.float32),
                              1.0)
            sum_det = jnp.sum(jnp.where(sel_c, dets[s], 0.0), axis=0,
                              keepdims=True)
            det_i = jnp.sum(jnp.where(r60_8 == i, dets[s], 0.0), axis=0,
                            keepdims=True)
            avg = sum_det / cnt
            avg = jnp.where(l8 == 0, 1.0, jnp.where(l8 == 1, m, avg))
            row = jnp.where(jnp.minimum(m_count, 7) > 1, avg, det_i)
            row = jnp.where(any_alive, row, neg1)
            nout.append(jnp.where(r20 == t, row, outs[s]))
            nar.append((alive_r & (~matched_r)).astype(jnp.float32))
            nac.append((alive_c & (~matched_c)).astype(jnp.float32))
        return tuple(nar), tuple(nac), tuple(nout)

    z20 = jnp.zeros((_NMS_TOPK, 8), jnp.float32)
    _, _, outs = lax.fori_loop(
        0, _NMS_TOPK, nms, (tuple(a_r0), tuple(a_c0), (z20,) * _B))

    pad = jnp.full((_TOPK - _NMS_TOPK, 8), -1.0, jnp.float32)
    for s in range(_B):
        out_ref[s] = jnp.concatenate([outs[s], pad], axis=0)


@jax.jit
def kernel(Cls, Shape, Offset):
    B = Cls.shape[0]
    cls3 = Cls.reshape(B, _ROWS, _LANES)
    shp4 = Shape.reshape(B, 3, _ROWS, _LANES)
    off4 = Offset.reshape(B, 3, _ROWS, _LANES)
    return pl.pallas_call(
        _body,
        out_shape=jax.ShapeDtypeStruct((B, _TOPK, 8), jnp.float32),
    )(cls3, shp4, off4)


# speedup vs baseline: 1.1342x; 1.0303x over previous
import jax
import jax.numpy as jnp
from jax import lax
from jax.experimental import pallas as pl
from jax.experimental.pallas import tpu as pltpu

_B = 16
_TOPK = 60
_THRESHOLD = 0.15
_NMS_THRESHOLD = 0.05
_NMS_TOPK = 20
_N = 24 * 24 * 24
_ROWS = 108
_LANES = 128
_NEG = -1e30
_HI = jax.lax.Precision.HIGHEST


def _body(cls_ref, shp_ref, off_ref, out_ref):
    riota = lax.broadcasted_iota(jnp.int32, (_ROWS, _LANES), 0)
    liota = lax.broadcasted_iota(jnp.int32, (_ROWS, _LANES), 1)
    fiota = riota * _LANES + liota
    k60c = lax.broadcasted_iota(jnp.int32, (_TOPK, 1), 0)
    r60 = lax.broadcasted_iota(jnp.int32, (_TOPK, _TOPK), 0)
    l60 = lax.broadcasted_iota(jnp.int32, (_TOPK, _TOPK), 1)
    eye = (r60 == l60).astype(jnp.float32)
    i60r = lax.broadcasted_iota(jnp.int32, (1, _TOPK), 1)
    r60_8 = lax.broadcasted_iota(jnp.int32, (_TOPK, 8), 0)
    r20 = lax.broadcasted_iota(jnp.int32, (_NMS_TOPK, 8), 0)
    l8 = lax.broadcasted_iota(jnp.int32, (1, 8), 1)
    zc = jnp.zeros((_TOPK, 1), jnp.float32)
    zi = jnp.zeros((_TOPK, 1), jnp.int32)
    neg1 = jnp.float32(-1.0)

    def tr(v):
        return lax.dot_general(v, eye, (((0,), (0,)), ((), ())),
                               precision=_HI)

    padV = jnp.full((20, _LANES), -1.0, jnp.float32)
    padI = jnp.full((20, _LANES), _N, jnp.int32)
    li128 = lax.broadcasted_iota(jnp.int32, (128, _LANES), 1)
    rowasc = lax.broadcasted_iota(jnp.int32, (128, 1), 0) >= 64

    def xshuf(x, j):
        li = lax.broadcasted_iota(jnp.int32, x.shape, 1)
        lo = pltpu.roll(x, _LANES - j, 1)
        hi = pltpu.roll(x, j, 1)
        return jnp.where((li & j) == 0, lo, hi)

    def stage(V, I, j, flip):
        pV = xshuf(V, j)
        pI = xshuf(I, j)
        R = (V > pV) | ((V == pV) & (I < pI))
        keep = R ^ flip
        return jnp.where(keep, V, pV), jnp.where(keep, I, pI)

    vals_t, idxs_t = [], []
    for s in range(_B):
        V = jnp.concatenate([1.0 / (1.0 + jnp.exp(-cls_ref[s])), padV],
                            axis=0)
        I = jnp.concatenate([fiota, padI], axis=0)
        for k in (2, 4, 8, 16, 32, 64, 128):
            asc = ((li128 & k) != 0) ^ rowasc
            jj = k // 2
            while jj >= 1:
                V, I = stage(V, I, jj, asc ^ ((li128 & jj) != 0))
                jj //= 2
        for h in (64, 32, 16, 8, 4, 2, 1):
            Av, Bv, Ai, Bi = V[:h], V[h:2 * h], I[:h], I[h:2 * h]
            R = (Av > Bv) | ((Av == Bv) & (Ai < Bi))
            V = jnp.where(R, Av, Bv)
            I = jnp.where(R, Ai, Bi)
            ra = lax.broadcasted_iota(jnp.int32, (h, 1), 0) >= ((h + 1) // 2)
            lih = lax.broadcasted_iota(jnp.int32, (h, _LANES), 1)
            for j2 in (64, 32, 16, 8, 4, 2, 1):
                V, I = stage(V, I, j2, ra ^ ((lih & j2) != 0))
        sv = V[:, :_TOPK]
        si = I[:, :_TOPK].astype(jnp.float32)
        vals_t.append(lax.dot_general(eye, sv, (((1,), (1,)), ((), ())),
                                      precision=_HI))
        idxs_t.append(lax.dot_general(eye, si, (((1,), (1,)), ((), ())),
                                      precision=_HI).astype(jnp.int32))

    dets, ious, ts_rs, ts_cs, bts, a_r0, a_c0 = [], [], [], [], [], [], []
    for s in range(_B):
        vals_c, idxs_c = vals_t[s], idxs_t[s]
        hi_c = idxs_c // _LANES
        lo_c = idxs_c % _LANES
        oh_hi = (hi_c == lax.broadcasted_iota(jnp.int32, (_TOPK, _ROWS), 1)
                 ).astype(jnp.float32)
        oh_lo = (lo_c == lax.broadcasted_iota(jnp.int32, (_TOPK, _LANES), 1)
                 ).astype(jnp.float32)

        planes = jnp.concatenate(
            [off_ref[s, 0], off_ref[s, 1], off_ref[s, 2],
             shp_ref[s, 0], shp_ref[s, 1], shp_ref[s, 2]], axis=1)
        rows_all = lax.dot_general(oh_hi, planes, (((1,), (0,)), ((), ())),
                                   precision=_HI)

        def pick(b):
            return jnp.sum(rows_all[:, b * 128:(b + 1) * 128] * oh_lo,
                           axis=1, keepdims=True)

        goz, goy, gox, gsz, gsy, gsx = (pick(b) for b in range(6))

        az = (idxs_c // 576).astype(jnp.float32)
        ay = ((idxs_c // 24) % 24).astype(jnp.float32)
        ax = (idxs_c % 24).astype(jnp.float32)

        cz = (az + goz) * 4.0
        cy = (ay + goy) * 4.0
        cx = (ax + gox) * 4.0
        sz = (2.0 * gsz) * 4.0
        sy = (2.0 * gsy) * 4.0
        sx = (2.0 * gsx) * 4.0

        valid_c = vals_c > _THRESHOLD
        det = jnp.concatenate(
            [jnp.where(valid_c, 1.0, neg1),
             jnp.where(valid_c, vals_c, neg1),
             jnp.where(valid_c, cz, neg1),
             jnp.where(valid_c, cy, neg1),
             jnp.where(valid_c, cx, neg1),
             jnp.where(valid_c, sz, neg1),
             jnp.where(valid_c, sy, neg1),
             jnp.where(valid_c, sx, neg1)], axis=1)

        vol_c = (jnp.maximum(sz, 0.0) * jnp.maximum(sy, 0.0)
                 ) * jnp.maximum(sx, 0.0)
        ts_c = vals_c
        validf = valid_c.astype(jnp.float32)

        colmat = jnp.concatenate(
            [cz, cy, cx, sz, sy, sx, vol_c, ts_c, validf], axis=1)
        rowmat = lax.dot_general(colmat, eye, (((0,), (0,)), ((), ())),
                                 precision=_HI)
        czr, cyr, cxr = rowmat[0:1], rowmat[1:2], rowmat[2:3]
        szr, syr, sxr = rowmat[3:4], rowmat[4:5], rowmat[5:6]
        vol_r, ts_r = rowmat[6:7], rowmat[7:8]

        def pair(c_c, s_c, c_r, s_r):
            lo_cc, hi_cc = c_c - s_c * 0.5, c_c + s_c * 0.5
            lo_rr, hi_rr = c_r - s_r * 0.5, c_r + s_r * 0.5
            return jnp.maximum(
                jnp.minimum(hi_cc, hi_rr) - jnp.maximum(lo_cc, lo_rr), 0.0)

        inter = (pair(cz, sz, czr, szr) * pair(cy, sy, cyr, syr)
                 ) * pair(cx, sx, cxr, sxr)
        union = (vol_c + vol_r) - inter
        iou = inter / jnp.maximum(union, 1e-6)
        mth = (iou >= _NMS_THRESHOLD).astype(jnp.float32)

        btT = ((ts_r > ts_c) | ((ts_r == ts_c) & (l60 < r60))
               ).astype(jnp.float32)

        dets.append(det)
        ious.append(mth)
        ts_rs.append(ts_r)
        ts_cs.append(ts_c)
        bts.append(btT)
        a_r0.append(rowmat[8:9])
        a_c0.append(validf)

    def nms(t, carry):
        ar, ac, outs = carry
        nar, nac, nout = [], [], []
        for s in range(_B):
            alive_r = ar[s] > 0.5
            alive_c = ac[s] > 0.5
            ms = jnp.where(alive_r, ts_rs[s], _NEG)
            m = jnp.max(ms)
            i = jnp.min(jnp.where(ms == m, i60r, _TOPK))
            any_alive = jnp.any(alive_r)
            mrow = jnp.sum(jnp.where(r60 == i, ious[s], 0.0), axis=0,
                           keepdims=True)
            mcol = jnp.sum(jnp.where(l60 == i, ious[s], 0.0), axis=1,
                           keepdims=True)
            matched_r = alive_r & (mrow > 0.5)
            matched_c = alive_c & (mcol > 0.5)
            m_count = jnp.sum(matched_r.astype(jnp.int32))
            rank_c = jnp.sum(bts[s] * matched_r.astype(jnp.float32),
                             axis=1, keepdims=True)
            sel_c = matched_c & (rank_c < 6.5)
            cnt = jnp.maximum(jnp.minimum(m_count, 7).astype(jnp.float32),
                              1.0)
            sum_det = jnp.sum(jnp.where(sel_c, dets[s], 0.0), axis=0,
                              keepdims=True)
            det_i = jnp.sum(jnp.where(r60_8 == i, dets[s], 0.0), axis=0,
                            keepdims=True)
            avg = sum_det / cnt
            avg = jnp.where(l8 == 0, 1.0, jnp.where(l8 == 1, m, avg))
            row = jnp.where(jnp.minimum(m_count, 7) > 1, avg, det_i)
            row = jnp.where(any_alive, row, neg1)
            nout.append(jnp.where(r20 == t, row, outs[s]))
            nar.append((alive_r & (~matched_r)).astype(jnp.float32))
            nac.append((alive_c & (~matched_c)).astype(jnp.float32))
        return tuple(nar), tuple(nac), tuple(nout)

    z20 = jnp.zeros((_NMS_TOPK, 8), jnp.float32)
    _, _, outs = lax.fori_loop(
        0, _NMS_TOPK, nms, (tuple(a_r0), tuple(a_c0), (z20,) * _B))

    pad = jnp.full((_TOPK - _NMS_TOPK, 8), -1.0, jnp.float32)
    for s in range(_B):
        out_ref[s] = jnp.concatenate([outs[s], pad], axis=0)


@jax.jit
def kernel(Cls, Shape, Offset):
    B = Cls.shape[0]
    cls3 = Cls.reshape(B, _ROWS, _LANES)
    shp4 = Shape.reshape(B, 3, _ROWS, _LANES)
    off4 = Offset.reshape(B, 3, _ROWS, _LANES)
    return pl.pallas_call(
        _body,
        out_shape=jax.ShapeDtypeStruct((B, _TOPK, 8), jnp.float32),
    )(cls3, shp4, off4)
